# Initial kernel scaffold; baseline (speedup 1.0000x reference)
import jax
import jax.numpy as jnp
from jax.experimental import pallas as pl


def kernel(x, table):
    raise NotImplementedError("write your pallas kernel here")


# speedup vs baseline: 158.8697x; 158.8697x over previous
import functools

import jax
import jax.numpy as jnp
from jax import lax
from jax.experimental import pallas as pl
from jax.experimental.pallas import tpu as pltpu, tpu_sc as plsc

L = 16
F = 2
LOG2_T = 19
T = 2 ** LOG2_T
BASE_RES = 16
PER_LEVEL_SCALE = 1.5
N = 1048576
P1 = 2654435761
P2 = 805459861

NC, NS = 2, 16
NW = NC * NS
PW = N // NW
C = 1024
NCHUNK = PW // C

_mesh = plsc.VectorSubcoreMesh(core_axis_name="c", subcore_axis_name="s")

_scratch = (
    [pltpu.VMEM((C,), jnp.float32) for _ in range(3)]
    + [pltpu.VMEM((C,), jnp.int32) for _ in range(8)]
    + [pltpu.VMEM((C,), jnp.float32) for _ in range(8)]
    + [pltpu.VMEM((C,), jnp.int32) for _ in range(8)]
    + [pltpu.VMEM((C,), jnp.float32) for _ in range(2)]
    + [pltpu.SemaphoreType.DMA]
)


@functools.partial(
    pl.kernel,
    out_type=jax.ShapeDtypeStruct((L, F, N), jnp.float32),
    mesh=_mesh,
    scratch_types=_scratch,
    compiler_params=pltpu.CompilerParams(needs_layout_passes=False),
)
def _hashgrid(x0, x1, x2, tbl, out, *refs):
    xv = refs[0:3]
    ibuf = refs[3:11]
    wbuf = refs[11:19]
    rbuf = refs[19:27]
    abuf = refs[27:29]
    sem = refs[29]

    wid = lax.axis_index("s") * NC + lax.axis_index("c")

    def chunk_body(ci, _):
        base = wid * PW + ci * C
        pltpu.sync_copy(x0.at[pl.ds(base, C)], xv[0])
        pltpu.sync_copy(x1.at[pl.ds(base, C)], xv[1])
        pltpu.sync_copy(x2.at[pl.ds(base, C)], xv[2])

        def level_body(l, scale):
            lvec = jnp.full((16,), l, jnp.int32)

            def grp_body(i, _):
                sl = pl.ds(i * 16, 16)
                px = ((xv[0][sl] + 1.0) * 0.5) * scale + 0.5
                py = ((xv[1][sl] + 1.0) * 0.5) * scale + 0.5
                pz = ((xv[2][sl] + 1.0) * 0.5) * scale + 0.5
                gx = px.astype(jnp.int32)
                gy = py.astype(jnp.int32)
                gz = pz.astype(jnp.int32)
                fx = px - gx.astype(jnp.float32)
                fy = py - gy.astype(jnp.float32)
                fz = pz - gz.astype(jnp.float32)
                hx0 = lax.bitcast_convert_type(gx, jnp.uint32)
                hx = (hx0, hx0 + jnp.uint32(1))
                hy0 = lax.bitcast_convert_type(gy, jnp.uint32) * jnp.uint32(P1)
                hy = (hy0, hy0 + jnp.uint32(P1))
                hz0 = lax.bitcast_convert_type(gz, jnp.uint32) * jnp.uint32(P2)
                hz = (hz0, hz0 + jnp.uint32(P2))
                wx = (1.0 - fx, fx)
                wy = (1.0 - fy, fy)
                wz = (1.0 - fz, fz)
                for corner in range(8):
                    bx, by, bz = corner & 1, (corner >> 1) & 1, (corner >> 2) & 1
                    h = (hx[bx] ^ hy[by] ^ hz[bz]) & jnp.uint32(T - 1)
                    row = lax.bitcast_convert_type(h, jnp.int32) | (lvec << LOG2_T)
                    ibuf[corner][sl] = row
                    wbuf[corner][sl] = wx[bx] * wy[by] * wz[bz]
                return 0

            lax.fori_loop(0, C // 16, grp_body, 0)

            copies = [pltpu.async_copy(tbl.at[ibuf[c]], rbuf[c], sem)
                      for c in range(8)]
            for cp in copies:
                cp.wait()

            def acc_body(j, _):
                sl = pl.ds(j * 16, 16)
                s0 = jnp.zeros((16,), jnp.float32)
                s1 = jnp.zeros((16,), jnp.float32)
                for c in range(8):
                    pair = plsc.bitcast(rbuf[c][sl], jnp.bfloat16)
                    f0, f1 = plsc.unpack(pair, format=plsc.PackFormat.INTERLEAVED)
                    wv = wbuf[c][sl]
                    s0 = s0 + f0 * wv
                    s1 = s1 + f1 * wv
                abuf[0][sl] = s0
                abuf[1][sl] = s1
                return 0

            lax.fori_loop(0, C // 16, acc_body, 0)
            pltpu.sync_copy(abuf[0], out.at[l, 0, pl.ds(base, C)])
            pltpu.sync_copy(abuf[1], out.at[l, 1, pl.ds(base, C)])
            return (scale + 1.0) * 1.5 - 1.0

        lax.fori_loop(0, L, level_body,
                      jnp.full((16,), float(BASE_RES - 1), jnp.float32))
        return 0

    lax.fori_loop(0, NCHUNK, chunk_body, 0)


def kernel(x, table):
    tb = table.astype(jnp.bfloat16)
    tu = lax.bitcast_convert_type(tb, jnp.uint16).astype(jnp.uint32)
    packed = (tu[..., 0] | (tu[..., 1] << 16)).reshape(L * T)
    packed = lax.bitcast_convert_type(packed, jnp.int32)
    out = _hashgrid(x[:, 0], x[:, 1], x[:, 2], packed)
    return out.transpose(2, 0, 1).reshape(N, L * F)


# speedup vs baseline: 175.9772x; 1.1077x over previous
import functools

import jax
import jax.numpy as jnp
from jax import lax
from jax.experimental import pallas as pl
from jax.experimental.pallas import tpu as pltpu, tpu_sc as plsc

L = 16
F = 2
LOG2_T = 19
T = 2 ** LOG2_T
BASE_RES = 16
PER_LEVEL_SCALE = 1.5
N = 1048576
P1 = 2654435761
P2 = 805459861

NC, NS = 2, 16
NW = NC * NS
PW = N // NW
C = 1024
NCHUNK = PW // C
TOT = NCHUNK * L

_mesh = plsc.VectorSubcoreMesh(core_axis_name="c", subcore_axis_name="s")

_scratch = (
    [pltpu.VMEM((C,), jnp.float32) for _ in range(3)]
    + [pltpu.VMEM((C,), jnp.int32) for _ in range(16)]
    + [pltpu.VMEM((C,), jnp.float32) for _ in range(16)]
    + [pltpu.VMEM((C,), jnp.int32) for _ in range(16)]
    + [pltpu.VMEM((C,), jnp.float32) for _ in range(2)]
    + [pltpu.SemaphoreType.DMA, pltpu.SemaphoreType.DMA]
)

_SCALE0 = float(BASE_RES - 1)


@functools.partial(
    pl.kernel,
    out_type=jax.ShapeDtypeStruct((L, F, N), jnp.float32),
    mesh=_mesh,
    scratch_types=_scratch,
    compiler_params=pltpu.CompilerParams(needs_layout_passes=False),
)
def _hashgrid(x0, x1, x2, tbl, out, *refs):
    xv = refs[0:3]
    ibuf = (refs[3:11], refs[11:19])
    wbuf = (refs[19:27], refs[27:35])
    rbuf = (refs[35:43], refs[43:51])
    abuf = refs[51:53]
    sems = refs[53:55]

    wid = lax.axis_index("s") * NC + lax.axis_index("c")

    def load_x(t):
        base = wid * PW + (t >> 4) * C
        pltpu.sync_copy(x0.at[pl.ds(base, C)], xv[0])
        pltpu.sync_copy(x1.at[pl.ds(base, C)], xv[1])
        pltpu.sync_copy(x2.at[pl.ds(base, C)], xv[2])

    def a2(t, scale, s):
        lvec = jnp.full((16,), t & (L - 1), jnp.int32)

        def grp_body(i, _):
            sl = pl.ds(i * 16, 16)
            px = ((xv[0][sl] + 1.0) * 0.5) * scale + 0.5
            py = ((xv[1][sl] + 1.0) * 0.5) * scale + 0.5
            pz = ((xv[2][sl] + 1.0) * 0.5) * scale + 0.5
            gx = px.astype(jnp.int32)
            gy = py.astype(jnp.int32)
            gz = pz.astype(jnp.int32)
            fx = px - gx.astype(jnp.float32)
            fy = py - gy.astype(jnp.float32)
            fz = pz - gz.astype(jnp.float32)
            hx0 = lax.bitcast_convert_type(gx, jnp.uint32)
            hx = (hx0, hx0 + jnp.uint32(1))
            hy0 = lax.bitcast_convert_type(gy, jnp.uint32) * jnp.uint32(P1)
            hy = (hy0, hy0 + jnp.uint32(P1))
            hz0 = lax.bitcast_convert_type(gz, jnp.uint32) * jnp.uint32(P2)
            hz = (hz0, hz0 + jnp.uint32(P2))
            wx = (1.0 - fx, fx)
            wy = (1.0 - fy, fy)
            wz = (1.0 - fz, fz)
            for corner in range(8):
                bx, by, bz = corner & 1, (corner >> 1) & 1, (corner >> 2) & 1
                h = (hx[bx] ^ hy[by] ^ hz[bz]) & jnp.uint32(T - 1)
                row = lax.bitcast_convert_type(h, jnp.int32) | (lvec << LOG2_T)
                ibuf[s][corner][sl] = row
                wbuf[s][corner][sl] = wx[bx] * wy[by] * wz[bz]
            return 0

        lax.fori_loop(0, C // 16, grp_body, 0)

    def fire(s):
        for c in range(8):
            pltpu.async_copy(tbl.at[ibuf[s][c]], rbuf[s][c], sems[s])

    def drain(s):
        for c in range(8):
            pltpu.make_async_copy(tbl.at[ibuf[s][c]], rbuf[s][c], sems[s]).wait()

    def acc_out(t, s):
        def acc_body(j, _):
            sl = pl.ds(j * 16, 16)
            s0 = jnp.zeros((16,), jnp.float32)
            s1 = jnp.zeros((16,), jnp.float32)
            for c in range(8):
                pair = plsc.bitcast(rbuf[s][c][sl], jnp.bfloat16)
                f0, f1 = plsc.unpack(pair, format=plsc.PackFormat.INTERLEAVED)
                wv = wbuf[s][c][sl]
                s0 = s0 + f0 * wv
                s1 = s1 + f1 * wv
            abuf[0][sl] = s0
            abuf[1][sl] = s1
            return 0

        lax.fori_loop(0, C // 16, acc_body, 0)
        l = t & (L - 1)
        base = wid * PW + (t >> 4) * C
        pltpu.sync_copy(abuf[0], out.at[l, 0, pl.ds(base, C)])
        pltpu.sync_copy(abuf[1], out.at[l, 1, pl.ds(base, C)])

    def half(t, scale, s):
        t1 = t + 1

        @pl.when(t1 < TOT)
        def _():
            @pl.when((t1 & (L - 1)) == 0)
            def _():
                load_x(t1)

            a2(t1, scale, 1 - s)
            fire(1 - s)

        drain(s)
        acc_out(t, s)
        return jnp.where((t1 & (L - 1)) == (L - 1),
                         jnp.full((16,), _SCALE0, jnp.float32),
                         (scale + 1.0) * 1.5 - 1.0)

    load_x(0)
    a2(0, jnp.full((16,), _SCALE0, jnp.float32), 0)
    fire(0)

    def body(k, scale):
        scale = half(2 * k, scale, 0)
        scale = half(2 * k + 1, scale, 1)
        return scale

    lax.fori_loop(0, TOT // 2, body,
                  jnp.full((16,), (_SCALE0 + 1.0) * 1.5 - 1.0, jnp.float32))


def kernel(x, table):
    tb = table.astype(jnp.bfloat16)
    tu = lax.bitcast_convert_type(tb, jnp.uint16).astype(jnp.uint32)
    packed = (tu[..., 0] | (tu[..., 1] << 16)).reshape(L * T)
    packed = lax.bitcast_convert_type(packed, jnp.int32)
    out = _hashgrid(x[:, 0], x[:, 1], x[:, 2], packed)
    return out.transpose(2, 0, 1).reshape(N, L * F)


# speedup vs baseline: 176.2300x; 1.0014x over previous
import functools

import jax
import jax.numpy as jnp
from jax import lax
from jax.experimental import pallas as pl
from jax.experimental.pallas import tpu as pltpu, tpu_sc as plsc

L = 16
F = 2
LOG2_T = 19
T = 2 ** LOG2_T
BASE_RES = 16
PER_LEVEL_SCALE = 1.5
N = 1048576
P1 = 2654435761
P2 = 805459861

NC, NS = 2, 16
NW = NC * NS
PW = N // NW
C = 1024
NCHUNK = PW // C
TOT = NCHUNK * L

_mesh = plsc.VectorSubcoreMesh(core_axis_name="c", subcore_axis_name="s")

_scratch = (
    [pltpu.VMEM((C,), jnp.float32) for _ in range(3)]
    + [pltpu.VMEM((C,), jnp.int32) for _ in range(16)]
    + [pltpu.VMEM((2 * C,), jnp.bfloat16) for _ in range(16)]
    + [pltpu.VMEM((C,), jnp.int32) for _ in range(16)]
    + [pltpu.VMEM((C,), jnp.float32) for _ in range(2)]
    + [pltpu.SemaphoreType.DMA, pltpu.SemaphoreType.DMA]
)

_SCALE0 = float(BASE_RES - 1)


@functools.partial(
    pl.kernel,
    out_type=jax.ShapeDtypeStruct((L, F, N), jnp.float32),
    mesh=_mesh,
    scratch_types=_scratch,
    compiler_params=pltpu.CompilerParams(needs_layout_passes=False),
)
def _hashgrid(x0, x1, x2, tbl, out, *refs):
    xv = refs[0:3]
    ibuf = (refs[3:11], refs[11:19])
    wbuf = (refs[19:27], refs[27:35])
    rbuf = (refs[35:43], refs[43:51])
    abuf = refs[51:53]
    sems = refs[53:55]

    wid = lax.axis_index("s") * NC + lax.axis_index("c")

    def load_x(t):
        base = wid * PW + (t >> 4) * C
        pltpu.sync_copy(x0.at[pl.ds(base, C)], xv[0])
        pltpu.sync_copy(x1.at[pl.ds(base, C)], xv[1])
        pltpu.sync_copy(x2.at[pl.ds(base, C)], xv[2])

        @plsc.parallel_loop(0, C // 16, unroll=2)
        def _(i):
            sl = pl.ds(i * 16, 16)
            for d in range(3):
                xv[d][sl] = (xv[d][sl] + 1.0) * 0.5

    def a2(t, scale, s):
        lsh = jnp.full((16,), t & (L - 1), jnp.int32) << LOG2_T

        @plsc.parallel_loop(0, C // 16, unroll=2)
        def _(i):
            sl = pl.ds(i * 16, 16)
            px = xv[0][sl] * scale + 0.5
            py = xv[1][sl] * scale + 0.5
            pz = xv[2][sl] * scale + 0.5
            gx = px.astype(jnp.int32)
            gy = py.astype(jnp.int32)
            gz = pz.astype(jnp.int32)
            fx = px - gx.astype(jnp.float32)
            fy = py - gy.astype(jnp.float32)
            fz = pz - gz.astype(jnp.float32)
            hx0 = lax.bitcast_convert_type(gx, jnp.uint32)
            hx = (hx0, hx0 + jnp.uint32(1))
            hy0 = lax.bitcast_convert_type(gy, jnp.uint32) * jnp.uint32(P1)
            hy = (hy0, hy0 + jnp.uint32(P1))
            hz0 = lax.bitcast_convert_type(gz, jnp.uint32) * jnp.uint32(P2)
            hz = (hz0, hz0 + jnp.uint32(P2))
            wx = (1.0 - fx, fx)
            wy = (1.0 - fy, fy)
            wz = (1.0 - fz, fz)
            sl2 = pl.ds(i * 32, 32)
            for corner in range(8):
                bx, by, bz = corner & 1, (corner >> 1) & 1, (corner >> 2) & 1
                h = (hx[bx] ^ hy[by] ^ hz[bz]) & jnp.uint32(T - 1)
                ibuf[s][corner][sl] = lax.bitcast_convert_type(h, jnp.int32) | lsh
                w = wx[bx] * wy[by] * wz[bz]
                wbuf[s][corner][sl2] = plsc.pack(
                    w, w, format=plsc.PackFormat.INTERLEAVED)

    def fire(s):
        for c in range(8):
            pltpu.async_copy(tbl.at[ibuf[s][c]], rbuf[s][c], sems[s])

    def drain(s):
        for c in range(8):
            pltpu.make_async_copy(tbl.at[ibuf[s][c]], rbuf[s][c], sems[s]).wait()

    def acc_out(t, s):
        @plsc.parallel_loop(0, C // 16, unroll=2)
        def _(j):
            sl = pl.ds(j * 16, 16)
            sl2 = pl.ds(j * 32, 32)
            sb = jnp.zeros((32,), jnp.bfloat16)
            for c in range(8):
                pair = plsc.bitcast(rbuf[s][c][sl], jnp.bfloat16)
                sb = sb + pair * wbuf[s][c][sl2]
            f0, f1 = plsc.unpack(sb, format=plsc.PackFormat.INTERLEAVED)
            abuf[0][sl] = f0
            abuf[1][sl] = f1

        l = t & (L - 1)
        base = wid * PW + (t >> 4) * C
        pltpu.sync_copy(abuf[0], out.at[l, 0, pl.ds(base, C)])
        pltpu.sync_copy(abuf[1], out.at[l, 1, pl.ds(base, C)])

    def half(t, scale, s):
        t1 = t + 1

        @pl.when(t1 < TOT)
        def _():
            @pl.when((t1 & (L - 1)) == 0)
            def _():
                load_x(t1)

            a2(t1, scale, 1 - s)
            fire(1 - s)

        drain(s)
        acc_out(t, s)
        return jnp.where((t1 & (L - 1)) == (L - 1),
                         jnp.full((16,), _SCALE0, jnp.float32),
                         (scale + 1.0) * 1.5 - 1.0)

    load_x(0)
    a2(0, jnp.full((16,), _SCALE0, jnp.float32), 0)
    fire(0)

    def body(k, scale):
        scale = half(2 * k, scale, 0)
        scale = half(2 * k + 1, scale, 1)
        return scale

    lax.fori_loop(0, TOT // 2, body,
                  jnp.full((16,), (_SCALE0 + 1.0) * 1.5 - 1.0, jnp.float32))


def kernel(x, table):
    tb = table.astype(jnp.bfloat16)
    tu = lax.bitcast_convert_type(tb, jnp.uint16).astype(jnp.uint32)
    packed = (tu[..., 0] | (tu[..., 1] << 16)).reshape(L * T)
    packed = lax.bitcast_convert_type(packed, jnp.int32)
    out = _hashgrid(x[:, 0], x[:, 1], x[:, 2], packed)
    return out.transpose(2, 0, 1).reshape(N, L * F)


# speedup vs baseline: 176.2383x; 1.0000x over previous
import functools

import jax
import jax.numpy as jnp
from jax import lax
from jax.experimental import pallas as pl
from jax.experimental.pallas import tpu as pltpu, tpu_sc as plsc

L = 16
F = 2
LOG2_T = 19
T = 2 ** LOG2_T
BASE_RES = 16
PER_LEVEL_SCALE = 1.5
N = 1048576
P1 = 2654435761
P2 = 805459861

NC, NS = 2, 16
NW = NC * NS
PW = N // NW
C = 1024
NCHUNK = PW // C
TOT = NCHUNK * L

_mesh = plsc.VectorSubcoreMesh(core_axis_name="c", subcore_axis_name="s")

_scratch = (
    [pltpu.VMEM((C,), jnp.float32) for _ in range(3)]
    + [pltpu.VMEM((C,), jnp.int32) for _ in range(16)]
    + [pltpu.VMEM((C,), jnp.int32) for _ in range(16)]
    + [pltpu.VMEM((C,), jnp.int32) for _ in range(16)]
    + [pltpu.VMEM((C,), jnp.float32) for _ in range(2)]
    + [pltpu.SemaphoreType.DMA, pltpu.SemaphoreType.DMA]
)

_SCALE0 = float(BASE_RES - 1)


@functools.partial(
    pl.kernel,
    out_type=jax.ShapeDtypeStruct((L, F, N), jnp.float32),
    mesh=_mesh,
    scratch_types=_scratch,
    compiler_params=pltpu.CompilerParams(needs_layout_passes=False),
)
def _hashgrid(x0, x1, x2, tbl, out, *refs):
    xv = refs[0:3]
    ibuf = (refs[3:11], refs[11:19])
    wbuf = (refs[19:27], refs[27:35])
    rbuf = (refs[35:43], refs[43:51])
    abuf = refs[51:53]
    sems = refs[53:55]

    wid = lax.axis_index("s") * NC + lax.axis_index("c")

    def load_x(t):
        base = wid * PW + (t >> 4) * C
        pltpu.sync_copy(x0.at[pl.ds(base, C)], xv[0])
        pltpu.sync_copy(x1.at[pl.ds(base, C)], xv[1])
        pltpu.sync_copy(x2.at[pl.ds(base, C)], xv[2])

        def xmap_body(i, _):
            sl = pl.ds(i * 16, 16)
            for d in range(3):
                xv[d][sl] = (xv[d][sl] + 1.0) * 0.5
            return 0

        lax.fori_loop(0, C // 16, xmap_body, 0)

    def a2(t, scale, s):
        lsh = jnp.full((16,), t & (L - 1), jnp.int32) << LOG2_T

        def grp_body(i, _):
            sl = pl.ds(i * 16, 16)
            px = xv[0][sl] * scale + 0.5
            py = xv[1][sl] * scale + 0.5
            pz = xv[2][sl] * scale + 0.5
            gx = px.astype(jnp.int32)
            gy = py.astype(jnp.int32)
            gz = pz.astype(jnp.int32)
            fx = px - gx.astype(jnp.float32)
            fy = py - gy.astype(jnp.float32)
            fz = pz - gz.astype(jnp.float32)
            hx0 = lax.bitcast_convert_type(gx, jnp.uint32)
            hx = (hx0, hx0 + jnp.uint32(1))
            hy0 = lax.bitcast_convert_type(gy, jnp.uint32) * jnp.uint32(P1)
            hy = (hy0, hy0 + jnp.uint32(P1))
            hz0 = lax.bitcast_convert_type(gz, jnp.uint32) * jnp.uint32(P2)
            hz = (hz0, hz0 + jnp.uint32(P2))
            wx = (1.0 - fx, fx)
            wy = (1.0 - fy, fy)
            wz = (1.0 - fz, fz)
            for corner in range(8):
                bx, by, bz = corner & 1, (corner >> 1) & 1, (corner >> 2) & 1
                h = (hx[bx] ^ hy[by] ^ hz[bz]) & jnp.uint32(T - 1)
                ibuf[s][corner][sl] = lax.bitcast_convert_type(h, jnp.int32) | lsh
                w = wx[bx] * wy[by] * wz[bz]
                wh = lax.shift_right_logical(
                    lax.bitcast_convert_type(w, jnp.uint32), jnp.uint32(16))
                wbuf[s][corner][sl] = lax.bitcast_convert_type(
                    wh | (wh << jnp.uint32(16)), jnp.int32)
            return 0

        lax.fori_loop(0, C // 16, grp_body, 0)

    def fire(s):
        for c in range(8):
            pltpu.async_copy(tbl.at[ibuf[s][c]], rbuf[s][c], sems[s])

    def drain(s):
        for c in range(8):
            pltpu.make_async_copy(tbl.at[ibuf[s][c]], rbuf[s][c], sems[s]).wait()

    def acc_out(t, s):
        def acc_body(j, _):
            sl = pl.ds(j * 16, 16)
            sb = jnp.zeros((32,), jnp.bfloat16)
            for c in range(8):
                pair = plsc.bitcast(rbuf[s][c][sl], jnp.bfloat16)
                wv = plsc.bitcast(wbuf[s][c][sl], jnp.bfloat16)
                sb = sb + pair * wv
            f0, f1 = plsc.unpack(sb, format=plsc.PackFormat.INTERLEAVED)
            abuf[0][sl] = f0
            abuf[1][sl] = f1
            return 0

        lax.fori_loop(0, C // 16, acc_body, 0)
        l = t & (L - 1)
        base = wid * PW + (t >> 4) * C
        pltpu.sync_copy(abuf[0], out.at[l, 0, pl.ds(base, C)])
        pltpu.sync_copy(abuf[1], out.at[l, 1, pl.ds(base, C)])

    def half(t, scale, s):
        t1 = t + 1

        @pl.when(t1 < TOT)
        def _():
            @pl.when((t1 & (L - 1)) == 0)
            def _():
                load_x(t1)

            a2(t1, scale, 1 - s)
            fire(1 - s)

        drain(s)
        acc_out(t, s)
        return jnp.where((t1 & (L - 1)) == (L - 1),
                         jnp.full((16,), _SCALE0, jnp.float32),
                         (scale + 1.0) * 1.5 - 1.0)

    load_x(0)
    a2(0, jnp.full((16,), _SCALE0, jnp.float32), 0)
    fire(0)

    def body(k, scale):
        scale = half(2 * k, scale, 0)
        scale = half(2 * k + 1, scale, 1)
        return scale

    lax.fori_loop(0, TOT // 2, body,
                  jnp.full((16,), (_SCALE0 + 1.0) * 1.5 - 1.0, jnp.float32))


def kernel(x, table):
    tb = table.astype(jnp.bfloat16)
    tu = lax.bitcast_convert_type(tb, jnp.uint16).astype(jnp.uint32)
    packed = (tu[..., 0] | (tu[..., 1] << 16)).reshape(L * T)
    packed = lax.bitcast_convert_type(packed, jnp.int32)
    out = _hashgrid(x[:, 0], x[:, 1], x[:, 2], packed)
    return out.transpose(2, 0, 1).reshape(N, L * F)


# speedup vs baseline: 218.3842x; 1.2391x over previous
import functools

import jax
import jax.numpy as jnp
from jax import lax
from jax.experimental import pallas as pl
from jax.experimental.pallas import tpu as pltpu, tpu_sc as plsc

L = 16
F = 2
LOG2_T = 19
T = 2 ** LOG2_T
BASE_RES = 16
PER_LEVEL_SCALE = 1.5
N = 1048576
P1 = 2654435761
P2 = 805459861

NC, NS = 2, 16
NW = NC * NS
PW = N // NW
C = 1024
NCHUNK = PW // C

SCALES = [float(BASE_RES * (PER_LEVEL_SCALE ** l) - 1.0) for l in range(L)]

ND = 3
_GS = [int(SCALES[l]) + 2 for l in range(ND)]
_GCAP = [-(-s * s * s // C) * C for s in _GS]
_GOFF = [sum(_GCAP[:i]) for i in range(ND)]
GRID_WORDS = sum(_GCAP)

SL = L - ND
TOT = NCHUNK * SL
_SCALE_D = float(SCALES[ND])

_mesh = plsc.VectorSubcoreMesh(core_axis_name="c", subcore_axis_name="s")

_scratch = (
    [pltpu.VMEM((C,), jnp.float32) for _ in range(3)]
    + [pltpu.VMEM((C,), jnp.int32) for _ in range(16)]
    + [pltpu.VMEM((C,), jnp.int32) for _ in range(16)]
    + [pltpu.VMEM((C,), jnp.int32) for _ in range(16)]
    + [pltpu.VMEM((C,), jnp.float32) for _ in range(2)]
    + [pltpu.VMEM((GRID_WORDS,), jnp.int32)]
    + [pltpu.SemaphoreType.DMA, pltpu.SemaphoreType.DMA]
)


def _wdup(w):
    wh = lax.shift_right_logical(
        lax.bitcast_convert_type(w, jnp.uint32), jnp.uint32(16))
    return lax.bitcast_convert_type(wh | (wh << jnp.uint32(16)), jnp.int32)


@functools.partial(
    pl.kernel,
    out_type=jax.ShapeDtypeStruct((L, F, N), jnp.float32),
    mesh=_mesh,
    scratch_types=_scratch,
    compiler_params=pltpu.CompilerParams(needs_layout_passes=False),
)
def _hashgrid(x0, x1, x2, tbl, out, *refs):
    xv = refs[0:3]
    ibuf = (refs[3:11], refs[11:19])
    wbuf = (refs[19:27], refs[27:35])
    rbuf = (refs[35:43], refs[43:51])
    abuf = refs[51:53]
    grid = refs[53]
    sems = refs[54:56]

    wid = lax.axis_index("s") * NC + lax.axis_index("c")
    iota = lax.iota(jnp.int32, 16)

    for dl in range(ND):
        S = _GS[dl]
        inv_s2 = 1.0 / float(S * S)
        inv_s = 1.0 / float(S)
        last = S * S * S - 1

        def build_batch(b, _, dl=dl, S=S, inv_s2=inv_s2, inv_s=inv_s, last=last):
            def idx_grp(i, _):
                lin = jnp.minimum(b * C + i * 16 + iota, last)
                lin_f = lin.astype(jnp.float32)
                x = ((lin_f + 0.5) * inv_s2).astype(jnp.int32)
                r = lin - x * (S * S)
                y = ((r.astype(jnp.float32) + 0.5) * inv_s).astype(jnp.int32)
                z = r - y * S
                h = (lax.bitcast_convert_type(x, jnp.uint32)
                     ^ (lax.bitcast_convert_type(y, jnp.uint32) * jnp.uint32(P1))
                     ^ (lax.bitcast_convert_type(z, jnp.uint32) * jnp.uint32(P2))
                     ) & jnp.uint32(T - 1)
                ibuf[0][0][pl.ds(i * 16, 16)] = (
                    lax.bitcast_convert_type(h, jnp.int32) | (dl << LOG2_T))
                return 0

            lax.fori_loop(0, C // 16, idx_grp, 0)
            pltpu.async_copy(
                tbl.at[ibuf[0][0]],
                grid.at[pl.ds(_GOFF[dl] + b * C, C)], sems[0]).wait()
            return 0

        lax.fori_loop(0, _GCAP[dl] // C, build_batch, 0)

    def load_x(t):
        base = wid * PW + (t // SL) * C
        pltpu.sync_copy(x0.at[pl.ds(base, C)], xv[0])
        pltpu.sync_copy(x1.at[pl.ds(base, C)], xv[1])
        pltpu.sync_copy(x2.at[pl.ds(base, C)], xv[2])

        def xmap_body(i, _):
            sl = pl.ds(i * 16, 16)
            for d in range(3):
                xv[d][sl] = (xv[d][sl] + 1.0) * 0.5
            return 0

        lax.fori_loop(0, C // 16, xmap_body, 0)

    def dense_levels(t):
        base = wid * PW + (t // SL) * C
        for dl in range(ND):
            S = _GS[dl]
            scale = SCALES[dl]

            def dgrp(i, _, S=S, scale=scale, off=_GOFF[dl]):
                sl = pl.ds(i * 16, 16)
                px = xv[0][sl] * scale + 0.5
                py = xv[1][sl] * scale + 0.5
                pz = xv[2][sl] * scale + 0.5
                gx = px.astype(jnp.int32)
                gy = py.astype(jnp.int32)
                gz = pz.astype(jnp.int32)
                fx = px - gx.astype(jnp.float32)
                fy = py - gy.astype(jnp.float32)
                fz = pz - gz.astype(jnp.float32)
                lin = (gx * S + gy) * S + gz + off
                wx = (1.0 - fx, fx)
                wy = (1.0 - fy, fy)
                wz = (1.0 - fz, fz)
                sb = jnp.zeros((32,), jnp.bfloat16)
                for corner in range(8):
                    bx, by, bz = corner & 1, (corner >> 1) & 1, (corner >> 2) & 1
                    v = plsc.load_gather(grid, [lin + (bx * S * S + by * S + bz)])
                    wv = _wdup(wx[bx] * wy[by] * wz[bz])
                    sb = sb + (plsc.bitcast(v, jnp.bfloat16)
                               * plsc.bitcast(wv, jnp.bfloat16))
                f0, f1 = plsc.unpack(sb, format=plsc.PackFormat.INTERLEAVED)
                abuf[0][sl] = f0
                abuf[1][sl] = f1
                return 0

            lax.fori_loop(0, C // 16, dgrp, 0)
            pltpu.sync_copy(abuf[0], out.at[dl, 0, pl.ds(base, C)])
            pltpu.sync_copy(abuf[1], out.at[dl, 1, pl.ds(base, C)])

    def a2(t, scale, s):
        lsh = jnp.full((16,), lax.rem(t, SL) + ND, jnp.int32) << LOG2_T

        def grp_body(i, _):
            sl = pl.ds(i * 16, 16)
            px = xv[0][sl] * scale + 0.5
            py = xv[1][sl] * scale + 0.5
            pz = xv[2][sl] * scale + 0.5
            gx = px.astype(jnp.int32)
            gy = py.astype(jnp.int32)
            gz = pz.astype(jnp.int32)
            fx = px - gx.astype(jnp.float32)
            fy = py - gy.astype(jnp.float32)
            fz = pz - gz.astype(jnp.float32)
            hx0 = lax.bitcast_convert_type(gx, jnp.uint32)
            hx = (hx0, hx0 + jnp.uint32(1))
            hy0 = lax.bitcast_convert_type(gy, jnp.uint32) * jnp.uint32(P1)
            hy = (hy0, hy0 + jnp.uint32(P1))
            hz0 = lax.bitcast_convert_type(gz, jnp.uint32) * jnp.uint32(P2)
            hz = (hz0, hz0 + jnp.uint32(P2))
            wx = (1.0 - fx, fx)
            wy = (1.0 - fy, fy)
            wz = (1.0 - fz, fz)
            for corner in range(8):
                bx, by, bz = corner & 1, (corner >> 1) & 1, (corner >> 2) & 1
                h = (hx[bx] ^ hy[by] ^ hz[bz]) & jnp.uint32(T - 1)
                ibuf[s][corner][sl] = lax.bitcast_convert_type(h, jnp.int32) | lsh
                wbuf[s][corner][sl] = _wdup(wx[bx] * wy[by] * wz[bz])
            return 0

        lax.fori_loop(0, C // 16, grp_body, 0)

    def fire(s):
        for c in range(8):
            pltpu.async_copy(tbl.at[ibuf[s][c]], rbuf[s][c], sems[s])

    def drain(s):
        for c in range(8):
            pltpu.make_async_copy(tbl.at[ibuf[s][c]], rbuf[s][c], sems[s]).wait()

    def acc_out(t, s):
        def acc_body(j, _):
            sl = pl.ds(j * 16, 16)
            sb = jnp.zeros((32,), jnp.bfloat16)
            for c in range(8):
                pair = plsc.bitcast(rbuf[s][c][sl], jnp.bfloat16)
                wv = plsc.bitcast(wbuf[s][c][sl], jnp.bfloat16)
                sb = sb + pair * wv
            f0, f1 = plsc.unpack(sb, format=plsc.PackFormat.INTERLEAVED)
            abuf[0][sl] = f0
            abuf[1][sl] = f1
            return 0

        lax.fori_loop(0, C // 16, acc_body, 0)
        l = lax.rem(t, SL) + ND
        base = wid * PW + (t // SL) * C
        pltpu.sync_copy(abuf[0], out.at[l, 0, pl.ds(base, C)])
        pltpu.sync_copy(abuf[1], out.at[l, 1, pl.ds(base, C)])

    def half(t, scale, s):
        t1 = t + 1

        @pl.when(t1 < TOT)
        def _():
            @pl.when(lax.rem(t1, SL) == 0)
            def _():
                load_x(t1)
                dense_levels(t1)

            a2(t1, scale, 1 - s)
            fire(1 - s)

        drain(s)
        acc_out(t, s)
        return jnp.where(lax.rem(t1, SL) == SL - 1,
                         jnp.full((16,), _SCALE_D, jnp.float32),
                         (scale + 1.0) * 1.5 - 1.0)

    load_x(0)
    dense_levels(0)
    a2(0, jnp.full((16,), _SCALE_D, jnp.float32), 0)
    fire(0)

    def body(k, scale):
        scale = half(2 * k, scale, 0)
        scale = half(2 * k + 1, scale, 1)
        return scale

    lax.fori_loop(0, TOT // 2, body,
                  jnp.full((16,), (_SCALE_D + 1.0) * 1.5 - 1.0, jnp.float32))


def kernel(x, table):
    tb = table.astype(jnp.bfloat16)
    tu = lax.bitcast_convert_type(tb, jnp.uint16).astype(jnp.uint32)
    packed = (tu[..., 0] | (tu[..., 1] << 16)).reshape(L * T)
    packed = lax.bitcast_convert_type(packed, jnp.int32)
    out = _hashgrid(x[:, 0], x[:, 1], x[:, 2], packed)
    return out.transpose(2, 0, 1).reshape(N, L * F)


# speedup vs baseline: 218.5042x; 1.0005x over previous
import functools

import jax
import jax.numpy as jnp
from jax import lax
from jax.experimental import pallas as pl
from jax.experimental.pallas import tpu as pltpu, tpu_sc as plsc

L = 16
F = 2
LOG2_T = 19
T = 2 ** LOG2_T
BASE_RES = 16
PER_LEVEL_SCALE = 1.5
N = 1048576
P1 = 2654435761
P2 = 805459861

NC, NS = 2, 16
NW = NC * NS
PW = N // NW
C = 1024
NCHUNK = PW // C

SCALES = [float(BASE_RES * (PER_LEVEL_SCALE ** l) - 1.0) for l in range(L)]

ND = 3
_GS = [int(SCALES[l]) + 2 for l in range(ND)]
_GCAP = [-(-s * s * s // C) * C for s in _GS]
_GOFF = [sum(_GCAP[:i]) for i in range(ND)]
GRID_WORDS = sum(_GCAP)

SL = L - ND
TOT = NCHUNK * SL
_SCALE_D = float(SCALES[ND])

_mesh = plsc.VectorSubcoreMesh(core_axis_name="c", subcore_axis_name="s")

_scratch = (
    [pltpu.VMEM((C,), jnp.float32) for _ in range(3)]
    + [pltpu.VMEM((C,), jnp.int32) for _ in range(16)]
    + [pltpu.VMEM((C,), jnp.float32) for _ in range(16)]
    + [pltpu.VMEM((C,), jnp.int32) for _ in range(16)]
    + [pltpu.VMEM((C,), jnp.float32) for _ in range(2)]
    + [pltpu.VMEM((GRID_WORDS,), jnp.int32)]
    + [pltpu.SemaphoreType.DMA, pltpu.SemaphoreType.DMA]
)


@functools.partial(
    pl.kernel,
    out_type=jax.ShapeDtypeStruct((L, F, N), jnp.float32),
    mesh=_mesh,
    scratch_types=_scratch,
    compiler_params=pltpu.CompilerParams(needs_layout_passes=False),
)
def _hashgrid(x0, x1, x2, tbl, out, *refs):
    xv = refs[0:3]
    ibuf = (refs[3:11], refs[11:19])
    wbuf = (refs[19:27], refs[27:35])
    rbuf = (refs[35:43], refs[43:51])
    abuf = refs[51:53]
    grid = refs[53]
    sems = refs[54:56]

    wid = lax.axis_index("s") * NC + lax.axis_index("c")
    iota = lax.iota(jnp.int32, 16)

    for dl in range(ND):
        S = _GS[dl]
        inv_s2 = 1.0 / float(S * S)
        inv_s = 1.0 / float(S)
        last = S * S * S - 1

        def build_batch(b, _, dl=dl, S=S, inv_s2=inv_s2, inv_s=inv_s, last=last):
            def idx_grp(i, _):
                lin = jnp.minimum(b * C + i * 16 + iota, last)
                lin_f = lin.astype(jnp.float32)
                x = ((lin_f + 0.5) * inv_s2).astype(jnp.int32)
                r = lin - x * (S * S)
                y = ((r.astype(jnp.float32) + 0.5) * inv_s).astype(jnp.int32)
                z = r - y * S
                h = (lax.bitcast_convert_type(x, jnp.uint32)
                     ^ (lax.bitcast_convert_type(y, jnp.uint32) * jnp.uint32(P1))
                     ^ (lax.bitcast_convert_type(z, jnp.uint32) * jnp.uint32(P2))
                     ) & jnp.uint32(T - 1)
                ibuf[0][0][pl.ds(i * 16, 16)] = (
                    lax.bitcast_convert_type(h, jnp.int32) | (dl << LOG2_T))
                return 0

            lax.fori_loop(0, C // 16, idx_grp, 0)
            pltpu.async_copy(
                tbl.at[ibuf[0][0]],
                grid.at[pl.ds(_GOFF[dl] + b * C, C)], sems[0]).wait()
            return 0

        lax.fori_loop(0, _GCAP[dl] // C, build_batch, 0)

    def load_x(t):
        base = wid * PW + (t // SL) * C
        pltpu.sync_copy(x0.at[pl.ds(base, C)], xv[0])
        pltpu.sync_copy(x1.at[pl.ds(base, C)], xv[1])
        pltpu.sync_copy(x2.at[pl.ds(base, C)], xv[2])

        def xmap_body(i, _):
            sl = pl.ds(i * 16, 16)
            for d in range(3):
                xv[d][sl] = (xv[d][sl] + 1.0) * 0.5
            return 0

        lax.fori_loop(0, C // 16, xmap_body, 0)

    def dense_levels(t):
        base = wid * PW + (t // SL) * C
        for dl in range(ND):
            S = _GS[dl]
            scale = SCALES[dl]

            def dgrp(i, _, S=S, scale=scale, off=_GOFF[dl]):
                sl = pl.ds(i * 16, 16)
                px = xv[0][sl] * scale + 0.5
                py = xv[1][sl] * scale + 0.5
                pz = xv[2][sl] * scale + 0.5
                gx = px.astype(jnp.int32)
                gy = py.astype(jnp.int32)
                gz = pz.astype(jnp.int32)
                fx = px - gx.astype(jnp.float32)
                fy = py - gy.astype(jnp.float32)
                fz = pz - gz.astype(jnp.float32)
                lin = (gx * S + gy) * S + gz + off
                wx = (1.0 - fx, fx)
                wy = (1.0 - fy, fy)
                wz = (1.0 - fz, fz)
                s0 = jnp.zeros((16,), jnp.float32)
                s1 = jnp.zeros((16,), jnp.float32)
                for corner in range(8):
                    bx, by, bz = corner & 1, (corner >> 1) & 1, (corner >> 2) & 1
                    v = plsc.load_gather(grid, [lin + (bx * S * S + by * S + bz)])
                    f0, f1 = plsc.unpack(plsc.bitcast(v, jnp.bfloat16),
                                         format=plsc.PackFormat.INTERLEAVED)
                    wv = wx[bx] * wy[by] * wz[bz]
                    s0 = s0 + f0 * wv
                    s1 = s1 + f1 * wv
                abuf[0][sl] = s0
                abuf[1][sl] = s1
                return 0

            lax.fori_loop(0, C // 16, dgrp, 0)
            pltpu.sync_copy(abuf[0], out.at[dl, 0, pl.ds(base, C)])
            pltpu.sync_copy(abuf[1], out.at[dl, 1, pl.ds(base, C)])

    def a2(t, scale, s):
        lsh = jnp.full((16,), lax.rem(t, SL) + ND, jnp.int32) << LOG2_T

        def grp_body(i, _):
            sl = pl.ds(i * 16, 16)
            px = xv[0][sl] * scale + 0.5
            py = xv[1][sl] * scale + 0.5
            pz = xv[2][sl] * scale + 0.5
            gx = px.astype(jnp.int32)
            gy = py.astype(jnp.int32)
            gz = pz.astype(jnp.int32)
            fx = px - gx.astype(jnp.float32)
            fy = py - gy.astype(jnp.float32)
            fz = pz - gz.astype(jnp.float32)
            hx0 = lax.bitcast_convert_type(gx, jnp.uint32)
            hx = (hx0, hx0 + jnp.uint32(1))
            hy0 = lax.bitcast_convert_type(gy, jnp.uint32) * jnp.uint32(P1)
            hy = (hy0, hy0 + jnp.uint32(P1))
            hz0 = lax.bitcast_convert_type(gz, jnp.uint32) * jnp.uint32(P2)
            hz = (hz0, hz0 + jnp.uint32(P2))
            wx = (1.0 - fx, fx)
            wy = (1.0 - fy, fy)
            wz = (1.0 - fz, fz)
            for corner in range(8):
                bx, by, bz = corner & 1, (corner >> 1) & 1, (corner >> 2) & 1
                h = (hx[bx] ^ hy[by] ^ hz[bz]) & jnp.uint32(T - 1)
                ibuf[s][corner][sl] = lax.bitcast_convert_type(h, jnp.int32) | lsh
                wbuf[s][corner][sl] = wx[bx] * wy[by] * wz[bz]
            return 0

        lax.fori_loop(0, C // 16, grp_body, 0)

    def fire(s):
        for c in range(8):
            pltpu.async_copy(tbl.at[ibuf[s][c]], rbuf[s][c], sems[s])

    def drain(s):
        for c in range(8):
            pltpu.make_async_copy(tbl.at[ibuf[s][c]], rbuf[s][c], sems[s]).wait()

    def acc_out(t, s):
        def acc_body(j, _):
            sl = pl.ds(j * 16, 16)
            s0 = jnp.zeros((16,), jnp.float32)
            s1 = jnp.zeros((16,), jnp.float32)
            for c in range(8):
                f0, f1 = plsc.unpack(plsc.bitcast(rbuf[s][c][sl], jnp.bfloat16),
                                     format=plsc.PackFormat.INTERLEAVED)
                wv = wbuf[s][c][sl]
                s0 = s0 + f0 * wv
                s1 = s1 + f1 * wv
            abuf[0][sl] = s0
            abuf[1][sl] = s1
            return 0

        lax.fori_loop(0, C // 16, acc_body, 0)
        l = lax.rem(t, SL) + ND
        base = wid * PW + (t // SL) * C
        pltpu.sync_copy(abuf[0], out.at[l, 0, pl.ds(base, C)])
        pltpu.sync_copy(abuf[1], out.at[l, 1, pl.ds(base, C)])

    def half(t, scale, s):
        t1 = t + 1

        @pl.when(t1 < TOT)
        def _():
            @pl.when(lax.rem(t1, SL) == 0)
            def _():
                load_x(t1)
                dense_levels(t1)

            a2(t1, scale, 1 - s)
            fire(1 - s)

        drain(s)
        acc_out(t, s)
        return jnp.where(lax.rem(t1, SL) == SL - 1,
                         jnp.full((16,), _SCALE_D, jnp.float32),
                         (scale + 1.0) * 1.5 - 1.0)

    load_x(0)
    dense_levels(0)
    a2(0, jnp.full((16,), _SCALE_D, jnp.float32), 0)
    fire(0)

    def body(k, scale):
        scale = half(2 * k, scale, 0)
        scale = half(2 * k + 1, scale, 1)
        return scale

    lax.fori_loop(0, TOT // 2, body,
                  jnp.full((16,), (_SCALE_D + 1.0) * 1.5 - 1.0, jnp.float32))


def kernel(x, table):
    tb = table.astype(jnp.bfloat16)
    tu = lax.bitcast_convert_type(tb, jnp.uint16).astype(jnp.uint32)
    packed = (tu[..., 0] | (tu[..., 1] << 16)).reshape(L * T)
    packed = lax.bitcast_convert_type(packed, jnp.int32)
    out = _hashgrid(x[:, 0], x[:, 1], x[:, 2], packed)
    return out.transpose(2, 0, 1).reshape(N, L * F)
